# baseline (speedup 1.0000x reference)
import jax
import jax.numpy as jnp
import numpy as np
from jax import lax
from jax.experimental import pallas as pl
from jax.experimental.pallas import tpu as pltpu
from jax.experimental.pallas import tpu_sc as plsc

K = 8
NC, NS, L = 2, 16, 16
NW = NC * NS
VG = 16
SG = 16

_CHOICE_128 = np.array([
    4, 2, 7, 1, 5, 3, 1, 7, 6, 2, 0, 2, 1, 3, 4, 2, 3, 2, 3, 7, 6, 3, 4, 3,
    4, 1, 0, 3, 4, 7, 5, 4, 5, 6, 3, 4, 6, 2, 1, 5, 7, 4, 7, 5, 1, 0, 4, 4,
    3, 5, 4, 3, 2, 3, 0, 7, 3, 2, 3, 1, 1, 6, 0, 0, 3, 1, 4, 0, 2, 1, 4, 5,
    3, 5, 4, 4, 6, 2, 1, 1, 7, 0, 5, 4, 1, 0, 0, 0, 0, 6, 7, 5, 6, 0, 3, 1,
    7, 7, 2, 1, 1, 4, 5, 4, 7, 4, 6, 2, 1, 1, 3, 7, 3, 4, 1, 3, 5, 0, 6, 3,
    3, 0, 2, 3, 2, 4, 6, 6,
], dtype=np.int32)


def _bfly(v, op):
    iota = lax.iota(jnp.int32, L)
    for sh in (8, 4, 2, 1):
        v = op(v, v.at[iota ^ sh].get(mode="promise_in_bounds"))
    return v


def _maxscan(ref, base, nvec):
    accs = [ref[pl.ds(base + a * L, L)] for a in range(4)]
    for j in range(4, nvec):
        accs[j % 4] = jnp.maximum(accs[j % 4], ref[pl.ds(base + j * L, L)])
    return jnp.maximum(
        jnp.maximum(accs[0], accs[1]), jnp.maximum(accs[2], accs[3])
    )


def _make_body(B, N, RPW, NGRP, NSG):
    NEG_INF = jnp.float32(-jnp.inf)

    def body(preds_hbm, choice_hbm, out_hbm, row_buf0, row_buf1, l1, l2,
             cbuf, rbuf, sem0, sem1):
        cid = lax.axis_index("c")
        sid = lax.axis_index("s")
        w = sid * NC + cid
        iota = lax.iota(jnp.int32, L)

        pltpu.sync_copy(choice_hbm.at[w], cbuf)
        cvec = cbuf[...]
        res = jnp.zeros((L,), jnp.int32)

        bufs = (row_buf0, row_buf1)
        sems = (sem0, sem1)
        cp = pltpu.async_copy(preds_hbm.at[w * RPW], bufs[0], sems[0])
        for r in range(RPW):
            row_buf = bufs[r % 2]
            cp.wait()
            if r + 1 < RPW:
                cp = pltpu.async_copy(
                    preds_hbm.at[w * RPW + r + 1],
                    bufs[(r + 1) % 2],
                    sems[(r + 1) % 2],
                )

            def p1(h, _, row_buf=row_buf):
                g = h * 2
                l1[pl.ds(g * L, L)] = _maxscan(row_buf, g * VG * L, VG)
                l1[pl.ds((g + 1) * L, L)] = _maxscan(
                    row_buf, (g + 1) * VG * L, VG
                )
                return 0

            lax.fori_loop(0, NGRP // 2, p1, 0)

            def p2(s, _):
                l2[pl.ds(s * L, L)] = _maxscan(l1, s * SG * L, SG)
                return 0

            lax.fori_loop(0, NSG, p2, 0)

            def ext(t, res, r=r, row_buf=row_buf):
                l2v = [l2[pl.ds(i * L, L)] for i in range(NSG)]
                m = l2v[0]
                for i in range(1, NSG):
                    m = jnp.maximum(m, l2v[i])
                mm = _bfly(m, jnp.maximum)
                c = jnp.full((L,), NSG, jnp.int32)
                for i in range(NSG):
                    c = jnp.minimum(c, jnp.where(l2v[i] == mm, i, NSG))
                istar = _bfly(c, jnp.minimum)[0]
                gb = istar * SG * L
                c2 = jnp.full((L,), SG, jnp.int32)
                for j in range(SG):
                    v = l1[pl.ds(gb + j * L, L)]
                    c2 = jnp.minimum(c2, jnp.where(v == mm, j, SG))
                g1 = istar * SG + _bfly(c2, jnp.minimum)[0]
                eb = g1 * VG * L
                best = jnp.full((L,), N, jnp.int32)
                for j in range(VG):
                    v = row_buf[pl.ds(eb + j * L, L)]
                    best = jnp.minimum(
                        best, jnp.where(v == mm, eb + j * L + iota, N)
                    )
                idx = _bfly(best, jnp.minimum)[0]
                res = jnp.where((iota == r) & (cvec == t), idx, res)
                voff = (idx // L) * L
                vv = row_buf[pl.ds(voff, L)]
                row_buf[pl.ds(voff, L)] = jnp.where(
                    iota == (idx - voff), NEG_INF, vv
                )
                l1[pl.ds(g1 * L, L)] = _maxscan(row_buf, eb, VG)
                l2[pl.ds(istar * L, L)] = _maxscan(l1, gb, SG)
                return res

            rounds = cvec[r] + 1
            res = lax.fori_loop(0, rounds, ext, res)

        rbuf[...] = res
        pltpu.sync_copy(rbuf, out_hbm.at[w])

    return body


def kernel(preds):
    assert preds.ndim == 2
    B, N = preds.shape
    RPW = B // NW
    NGRP = N // (VG * L)
    NSG = NGRP // SG
    assert B % NW == 0 and N % (VG * L * SG) == 0 and NGRP % 2 == 0

    if B == 128:
        choice2d_np = np.zeros((NW, L), np.int32)
        choice2d_np[:, :RPW] = _CHOICE_128.reshape(NW, RPW)
        choice2d = jnp.asarray(choice2d_np)
    else:
        choice = jax.random.randint(jax.random.key(42), (B,), 0, K)
        choice2d = jnp.zeros((NW, L), jnp.int32).at[:, :RPW].set(
            choice.reshape(NW, RPW).astype(jnp.int32)
        )

    f = pl.kernel(
        _make_body(B, N, RPW, NGRP, NSG),
        out_type=jax.ShapeDtypeStruct((NW, L), jnp.int32),
        mesh=plsc.VectorSubcoreMesh(
            core_axis_name="c", subcore_axis_name="s",
            num_cores=NC, num_subcores=NS,
        ),
        scratch_types=[
            pltpu.VMEM((N,), jnp.float32),
            pltpu.VMEM((N,), jnp.float32),
            pltpu.VMEM((NGRP * L,), jnp.float32),
            pltpu.VMEM((NSG * L,), jnp.float32),
            pltpu.VMEM((L,), jnp.int32),
            pltpu.VMEM((L,), jnp.int32),
            pltpu.SemaphoreType.DMA,
            pltpu.SemaphoreType.DMA,
        ],
    )
    out2d = f(preds, choice2d)
    return out2d[:, :RPW].reshape(B)


# speedup vs baseline: 1.1174x; 1.1174x over previous
import jax
import jax.numpy as jnp
import numpy as np
from jax import lax
from jax.experimental import pallas as pl
from jax.experimental.pallas import tpu as pltpu
from jax.experimental.pallas import tpu_sc as plsc

K = 8
NC, NS, L = 2, 16, 16
NW = NC * NS

_CHOICE_128 = np.array([
    4, 2, 7, 1, 5, 3, 1, 7, 6, 2, 0, 2, 1, 3, 4, 2, 3, 2, 3, 7, 6, 3, 4, 3,
    4, 1, 0, 3, 4, 7, 5, 4, 5, 6, 3, 4, 6, 2, 1, 5, 7, 4, 7, 5, 1, 0, 4, 4,
    3, 5, 4, 3, 2, 3, 0, 7, 3, 2, 3, 1, 1, 6, 0, 0, 3, 1, 4, 0, 2, 1, 4, 5,
    3, 5, 4, 4, 6, 2, 1, 1, 7, 0, 5, 4, 1, 0, 0, 0, 0, 6, 7, 5, 6, 0, 3, 1,
    7, 7, 2, 1, 1, 4, 5, 4, 7, 4, 6, 2, 1, 1, 3, 7, 3, 4, 1, 3, 5, 0, 6, 3,
    3, 0, 2, 3, 2, 4, 6, 6,
], dtype=np.int32)


def _bfly(v, op):
    iota = lax.iota(jnp.int32, L)
    for sh in (8, 4, 2, 1):
        v = op(v, v.at[iota ^ sh].get(mode="promise_in_bounds"))
    return v


def _make_body(B, N, RPW, NBLK, BLK):
    NEG_INF = jnp.float32(-jnp.inf)
    NG = NBLK // L

    def body(preds_hbm, choice_hbm, out_hbm, row_buf0, row_buf1, cbuf, rbuf,
             sem0, sem1):
        cid = lax.axis_index("c")
        sid = lax.axis_index("s")
        w = sid * NC + cid
        iota = lax.iota(jnp.int32, L)

        pltpu.sync_copy(choice_hbm.at[w], cbuf)
        cvec = cbuf[...]
        res = jnp.zeros((L,), jnp.int32)

        bufs = (row_buf0, row_buf1)
        sems = (sem0, sem1)
        cp = pltpu.async_copy(preds_hbm.at[w * RPW], bufs[0], sems[0])
        for r in range(RPW):
            row_buf = bufs[r % 2]
            cp.wait()
            if r + 1 < RPW:
                cp = pltpu.async_copy(
                    preds_hbm.at[w * RPW + r + 1],
                    bufs[(r + 1) % 2],
                    sems[(r + 1) % 2],
                )

            def p1(blk, bms, row_buf=row_buf):
                base = blk * BLK
                accs = [row_buf[pl.ds(base + a * L, L)] for a in range(4)]
                for j in range(4, BLK // L):
                    accs[j % 4] = jnp.maximum(
                        accs[j % 4], row_buf[pl.ds(base + j * L, L)]
                    )
                acc = jnp.maximum(
                    jnp.maximum(accs[0], accs[1]),
                    jnp.maximum(accs[2], accs[3]),
                )
                m = _bfly(acc, jnp.maximum)
                return tuple(
                    jnp.where(iota + g * L == blk, m, bms[g]) for g in range(NG)
                )

            bms = lax.fori_loop(
                0, NBLK, p1,
                tuple(jnp.full((L,), NEG_INF, jnp.float32) for _ in range(NG)),
            )

            def ext(t, carry, r=r, row_buf=row_buf):
                bm0, bm1, bm2, bm3, res = carry
                mm = _bfly(
                    jnp.maximum(jnp.maximum(bm0, bm1), jnp.maximum(bm2, bm3)),
                    jnp.maximum,
                )
                kcand = jnp.full((L,), NBLK, jnp.int32)
                for i, bmi in enumerate((bm0, bm1, bm2, bm3)):
                    kcand = jnp.minimum(
                        kcand, jnp.where(bmi == mm, iota + i * L, NBLK)
                    )
                kstar = _bfly(kcand, jnp.minimum)[0]
                base = kstar * BLK
                bests = [jnp.full((L,), N, jnp.int32) for _ in range(4)]
                for j in range(BLK // L):
                    v = row_buf[pl.ds(base + j * L, L)]
                    bests[j % 4] = jnp.minimum(
                        bests[j % 4], jnp.where(v == mm, base + j * L + iota, N)
                    )
                best = jnp.minimum(
                    jnp.minimum(bests[0], bests[1]),
                    jnp.minimum(bests[2], bests[3]),
                )
                idx = _bfly(best, jnp.minimum)[0]
                res = jnp.where((iota == r) & (cvec == t), idx, res)
                voff = (idx // L) * L
                vv = row_buf[pl.ds(voff, L)]
                row_buf[pl.ds(voff, L)] = jnp.where(
                    iota == (idx - voff), NEG_INF, vv
                )
                accs = [row_buf[pl.ds(base + a * L, L)] for a in range(4)]
                for j in range(4, BLK // L):
                    accs[j % 4] = jnp.maximum(
                        accs[j % 4], row_buf[pl.ds(base + j * L, L)]
                    )
                nb = _bfly(
                    jnp.maximum(
                        jnp.maximum(accs[0], accs[1]),
                        jnp.maximum(accs[2], accs[3]),
                    ),
                    jnp.maximum,
                )
                bm0 = jnp.where(iota + 0 * L == kstar, nb, bm0)
                bm1 = jnp.where(iota + 1 * L == kstar, nb, bm1)
                bm2 = jnp.where(iota + 2 * L == kstar, nb, bm2)
                bm3 = jnp.where(iota + 3 * L == kstar, nb, bm3)
                return bm0, bm1, bm2, bm3, res

            rounds = cvec[r] + 1
            carry = (bms[0], bms[1], bms[2], bms[3], res)
            res = lax.fori_loop(0, rounds, ext, carry)[4]

        rbuf[...] = res
        pltpu.sync_copy(rbuf, out_hbm.at[w])

    return body


def kernel(preds):
    assert preds.ndim == 2
    B, N = preds.shape
    RPW = B // NW
    BLK = 512
    NBLK = N // BLK
    assert B % NW == 0 and N % BLK == 0 and NBLK == 4 * L and BLK % L == 0

    if B == 128:
        choice2d_np = np.zeros((NW, L), np.int32)
        choice2d_np[:, :RPW] = _CHOICE_128.reshape(NW, RPW)
        choice2d = jnp.asarray(choice2d_np)
    else:
        choice = jax.random.randint(jax.random.key(42), (B,), 0, K)
        choice2d = jnp.zeros((NW, L), jnp.int32).at[:, :RPW].set(
            choice.reshape(NW, RPW).astype(jnp.int32)
        )

    f = pl.kernel(
        _make_body(B, N, RPW, NBLK, BLK),
        out_type=jax.ShapeDtypeStruct((NW, L), jnp.int32),
        mesh=plsc.VectorSubcoreMesh(
            core_axis_name="c", subcore_axis_name="s",
            num_cores=NC, num_subcores=NS,
        ),
        scratch_types=[
            pltpu.VMEM((N,), jnp.float32),
            pltpu.VMEM((N,), jnp.float32),
            pltpu.VMEM((L,), jnp.int32),
            pltpu.VMEM((L,), jnp.int32),
            pltpu.SemaphoreType.DMA,
            pltpu.SemaphoreType.DMA,
        ],
    )
    out2d = f(preds, choice2d)
    return out2d[:, :RPW].reshape(B)
